# Initial kernel scaffold; baseline (speedup 1.0000x reference)
import jax
import jax.numpy as jnp
from jax.experimental import pallas as pl


def kernel(pred, s_coords, s_values):
    raise NotImplementedError("write your pallas kernel here")


# speedup vs baseline: 1.4292x; 1.4292x over previous
import jax
import jax.numpy as jnp
from jax import lax
from jax.experimental import pallas as pl
from jax.experimental.pallas import tpu as pltpu
from jax.experimental.pallas import tpu_sc as plsc

_LAMBDA_POINT = 20.0

_B, _T, _H, _W = 8, 16, 256, 256
_N = 512
_BT = _B * _T
_NC, _NS, _L = 2, 16, 16
_NW = _NC * _NS
_BT_PER_W = _BT // _NW
_PTS_PER_W = _BT_PER_W * _N
_CHUNK = 128
_NCHUNK = _PTS_PER_W // _CHUNK


def _point_loss_sc(pred_hbm, idx_hbm, sv_hbm, out_hbm,
                   idx_v, g_v, sv_v, acc_v, sem):
    cid = lax.axis_index("c")
    sid = lax.axis_index("s")
    wid = cid * _NS + sid

    pltpu.sync_copy(idx_hbm.at[wid], idx_v)
    pltpu.sync_copy(sv_hbm.at[pl.ds(wid * _PTS_PER_W, _PTS_PER_W)], sv_v)

    copies = []
    for j in range(_NCHUNK):
        copies.append(
            pltpu.async_copy(pred_hbm.at[idx_v.at[j]],
                             g_v.at[pl.ds(j * _CHUNK, _CHUNK)], sem))
    for c in copies:
        c.wait()

    def accum(i, acc):
        sl = pl.ds(i * _L, _L)
        d = g_v[sl] - sv_v[sl]
        return acc + d * d

    acc = lax.fori_loop(0, _PTS_PER_W // _L, accum,
                        jnp.zeros((_L,), jnp.float32))
    acc_v[...] = acc * (_LAMBDA_POINT / (_BT * _N))
    pltpu.sync_copy(acc_v, out_hbm.at[wid])


@jax.jit
def kernel(pred, s_coords, s_values):
    pred_flat = pred.reshape(-1)
    rows = s_coords[:, 0].astype(jnp.int32)
    cols = s_coords[:, 1].astype(jnp.int32)
    sv = s_values.reshape(-1).astype(jnp.float32)

    flat = rows * _W + cols
    g_idx = (jnp.arange(_BT, dtype=jnp.int32) * (_H * _W))[:, None] + flat[None, :]
    idx_all = g_idx.reshape(_NW, _NCHUNK, _CHUNK)

    mesh = plsc.VectorSubcoreMesh(core_axis_name="c", subcore_axis_name="s")
    f = pl.kernel(
        _point_loss_sc,
        mesh=mesh,
        out_type=jax.ShapeDtypeStruct((_NW, _L), jnp.float32),
        scratch_types=[
            pltpu.VMEM((_NCHUNK, _CHUNK), jnp.int32),
            pltpu.VMEM((_PTS_PER_W,), jnp.float32),
            pltpu.VMEM((_PTS_PER_W,), jnp.float32),
            pltpu.VMEM((_L,), jnp.float32),
            pltpu.SemaphoreType.DMA,
        ],
    )
    partial = f(pred_flat, idx_all, sv)
    return jnp.sum(partial)
